# Initial kernel scaffold; baseline (speedup 1.0000x reference)
import jax
import jax.numpy as jnp
from jax.experimental import pallas as pl


def kernel(x, edge_index, batch, Wq1, bq1, Wk1, bk1, Wv1, bv1, Ws1, bs1, Wq2, bq2, Wk2, bk2, Wv2, bv2, Ws2, bs2, Wfc, bfc):
    raise NotImplementedError("write your pallas kernel here")


# speedup vs baseline: 7.0549x; 7.0549x over previous
import functools
import math

import jax
import jax.numpy as jnp
from jax import lax
from jax.experimental import pallas as pl
from jax.experimental.pallas import tpu as pltpu
from jax.experimental.pallas import tpu_sc as plsc

N = 100000
E = 1600000
G = 64
HALF = N // 2

C1 = 256
C2 = 256
EPT32 = ((E // 32 + C1 - 1) // C1) * C1
EPT16 = ((E // 16 + C2 - 1) // C2) * C2
E_PAD = max(32 * EPT32, 16 * EPT16)


def _edge_pass(dq, edge_split, C, dbg=4):
    SUB = C // 128
    inv_sqrt = 1.0 / math.sqrt(dq)
    if edge_split:
        n_chunks = EPT32 // C
        irows_per_tile = EPT32 // 128
        agg_rows = N
    else:
        n_chunks = EPT16 // C
        irows_per_tile = EPT16 // 128
        agg_rows = HALF
    tile_rows = agg_rows // 16
    den_tile = (agg_rows // 8 + 15) // 16
    den_rows = 16 * den_tile
    nfull, rem = divmod(tile_rows, C)
    nfulld, remd = divmod(den_tile, C)
    if edge_split:
        out_type = [jax.ShapeDtypeStruct((2, 16, tile_rows, dq), jnp.float32),
                    jax.ShapeDtypeStruct((2, 16, den_tile, 8), jnp.float32)]
    else:
        out_type = [jax.ShapeDtypeStruct((32, tile_rows, dq), jnp.float32),
                    jax.ShapeDtypeStruct((32, den_tile, 8), jnp.float32)]
    zoffs = list(range(0, dq, 16))

    mesh = plsc.VectorSubcoreMesh(core_axis_name="c", subcore_axis_name="s")

    @functools.partial(
        pl.kernel,
        out_type=out_type,
        mesh=mesh,
        compiler_params=pltpu.CompilerParams(needs_layout_passes=False,
                                             use_tc_tiling_on_sc=False),
        scratch_types=[
            pltpu.VMEM((SUB, 128), jnp.int32),
            pltpu.VMEM((SUB, 128), jnp.int32),
            pltpu.VMEM((SUB, 128), jnp.int32),
            pltpu.VMEM((C, dq), jnp.float32),
            pltpu.VMEM((C, dq), jnp.float32),
            pltpu.VMEM((C,), jnp.float32),
            pltpu.VMEM((C, 8), jnp.float32),
            pltpu.VMEM_SHARED((agg_rows, dq), jnp.float32),
            pltpu.VMEM_SHARED((den_rows, 8), jnp.float32),
            pltpu.SemaphoreType.DMA,
            pltpu.SemaphoreType.DMA,
        ],
    )
    def run(srcp, dstp, qt, kt, vt, outa, outd,
            sidx, didx, dnidx, qrows, krows, tbuf, denbuf, agg, den,
            semg, sems):
        cid = lax.axis_index("c")
        sid = lax.axis_index("s")
        z = jnp.zeros((16,), jnp.float32)
        iot = lax.iota(jnp.int32, 16)

        def zrow(r, carry):
            for o in zoffs:
                qrows[r, pl.ds(o, 16)] = z
            return carry

        lax.fori_loop(0, C, zrow, 0)

        def zden(k, carry):
            flat = k * 16 + iot
            plsc.store_scatter(denbuf,
                               [lax.shift_right_logical(flat, 3),
                                lax.bitwise_and(flat, 7)], z)
            return carry

        lax.fori_loop(0, C // 2, zden, 0)

        zbase = sid * tile_rows
        for tz in range(nfull):
            pltpu.sync_copy(qrows.at[pl.ds(0, C)],
                            agg.at[pl.ds(zbase + tz * C, C)])
        if rem:
            pltpu.sync_copy(qrows.at[pl.ds(0, rem)],
                            agg.at[pl.ds(zbase + nfull * C, rem)])
        dzbase = sid * den_tile
        for tz in range(nfulld):
            pltpu.sync_copy(denbuf.at[pl.ds(0, C)],
                            den.at[pl.ds(dzbase + tz * C, C)])
        if remd:
            pltpu.sync_copy(denbuf.at[pl.ds(0, remd)],
                            den.at[pl.ds(dzbase + nfulld * C, remd)])
        plsc.subcore_barrier()

        if edge_split:
            wid = sid * 2 + cid
            tile_ebase = wid * EPT32
            tile_irow = wid * irows_per_tile
            lo = 0
        else:
            tile_ebase = sid * EPT16
            tile_irow = sid * irows_per_tile
            lo = cid * HALF

        def chunk(g, carry):
            ebase = tile_ebase + g * C
            irow = tile_irow + g * SUB
            pltpu.sync_copy(srcp.at[pl.ds(irow, SUB)], sidx)
            pltpu.sync_copy(dstp.at[pl.ds(irow, SUB)], didx)
            if dbg >= 2:
                descs = []
                for m in range(SUB):
                    sl = pl.ds(m * 128, 128)
                    descs.append(pltpu.async_copy(qt.at[didx.at[m]],
                                                  qrows.at[sl], semg))
                    descs.append(pltpu.async_copy(kt.at[sidx.at[m]],
                                                  krows.at[sl], semg))
                for dsc in descs:
                    dsc.wait()

            def dot_grp(j, c2):
                rows = j * 16 + iot
                acc = jnp.zeros((16,), jnp.float32)
                for i in range(dq):
                    ci = jnp.full((16,), i, jnp.int32)
                    acc = acc + (plsc.load_gather(qrows, [rows, ci]) *
                                 plsc.load_gather(krows, [rows, ci]))
                jj = j // 8
                jl = (j % 8) * 16
                dv = didx[jj, pl.ds(jl, 16)]
                ok = (ebase + rows) < E
                if not edge_split:
                    ok = ok & (dv >= lo) & (dv < lo + HALF)
                t = jnp.where(ok, jnp.exp(acc * inv_sqrt), 0.0)
                tbuf[pl.ds(j * 16, 16)] = t
                dloc = jnp.where(ok, dv - lo, 0)
                didx[jj, pl.ds(jl, 16)] = dloc
                dnidx[jj, pl.ds(jl, 16)] = lax.shift_right_logical(dloc, 3)
                return c2

            if dbg >= 3:
                lax.fori_loop(0, C // 16, dot_grp, 0)
                vdescs = []
                for m in range(SUB):
                    vdescs.append(pltpu.async_copy(vt.at[sidx.at[m]],
                                                   qrows.at[pl.ds(m * 128,
                                                                  128)],
                                                   semg))
                for dsc in vdescs:
                    dsc.wait()

            def scale_grp(j, c2):
                rows = j * 16 + iot
                jj = j // 8
                jl = (j % 8) * 16
                t = tbuf[pl.ds(j * 16, 16)]
                for i in range(dq):
                    ci = jnp.full((16,), i, jnp.int32)
                    vv = plsc.load_gather(qrows, [rows, ci])
                    plsc.store_scatter(qrows, [rows, ci], vv * t)
                dloc = didx[jj, pl.ds(jl, 16)]
                plsc.store_scatter(denbuf,
                                   [rows, lax.bitwise_and(dloc, 7)], t)
                return c2

            if dbg >= 4:
                lax.fori_loop(0, C // 16, scale_grp, 0)
                sdescs = []
                for m in range(SUB):
                    sl = pl.ds(m * 128, 128)
                    sdescs.append(pltpu.async_copy(qrows.at[sl],
                                                   agg.at[didx.at[m]],
                                                   sems, add=True))
                    sdescs.append(pltpu.async_copy(denbuf.at[sl],
                                                   den.at[dnidx.at[m]],
                                                   sems, add=True))
                for dsc in sdescs:
                    dsc.wait()
                lax.fori_loop(0, C // 2, zden, 0)
            return carry

        if dbg >= 1:
            lax.fori_loop(0, n_chunks, chunk, 0)

        plsc.subcore_barrier()
        if edge_split:
            pltpu.sync_copy(agg.at[pl.ds(sid * tile_rows, tile_rows)],
                            outa.at[cid, sid])
            pltpu.sync_copy(den.at[pl.ds(sid * den_tile, den_tile)],
                            outd.at[cid, sid])
        else:
            pltpu.sync_copy(agg.at[pl.ds(sid * tile_rows, tile_rows)],
                            outa.at[cid * 16 + sid])
            pltpu.sync_copy(den.at[pl.ds(sid * den_tile, den_tile)],
                            outd.at[cid * 16 + sid])

    return run


_edge1 = _edge_pass(16, True, C1)
_edge2 = _edge_pass(32, False, C2)

_R = 1000


def _proj1(x, w1, b1):
    def body(x_ref, w_ref, b_ref, o_ref):
        o_ref[...] = (jnp.dot(x_ref[...], w_ref[...],
                              preferred_element_type=jnp.float32) + b_ref[...])

    return pl.pallas_call(
        body,
        grid=(N // _R,),
        in_specs=[pl.BlockSpec((_R, 9), lambda i: (i, 0)),
                  pl.BlockSpec((9, 48), lambda i: (0, 0)),
                  pl.BlockSpec((1, 48), lambda i: (0, 0))],
        out_specs=pl.BlockSpec((_R, 48), lambda i: (i, 0)),
        out_shape=jax.ShapeDtypeStruct((N, 48), jnp.float32),
    )(x, w1, b1)


def _mid(e1a, den1, x, ws1, bs1, w2, b2, ws2, bs2):
    def body(a_ref, d_ref, x_ref, ws1_ref, bs1_ref, w2_ref, b2_ref, ws2_ref,
             bs2_ref, o_ref):
        aggv = a_ref[0] + a_ref[1]
        den = d_ref[0] + d_ref[1]
        s1 = (jnp.dot(x_ref[...], ws1_ref[...],
                      preferred_element_type=jnp.float32) + bs1_ref[...])
        h = jnp.maximum(aggv / jnp.maximum(den, 1e-16) + s1, 0.0)
        qkv = (jnp.dot(h, w2_ref[...], preferred_element_type=jnp.float32)
               + b2_ref[...])
        s2 = (jnp.dot(h, ws2_ref[...], preferred_element_type=jnp.float32)
              + bs2_ref[...])
        o_ref[...] = jnp.concatenate([qkv, s2], axis=1)

    return pl.pallas_call(
        body,
        grid=(N // _R,),
        in_specs=[pl.BlockSpec((2, _R, 16), lambda i: (0, i, 0)),
                  pl.BlockSpec((2, _R, 1), lambda i: (0, i, 0)),
                  pl.BlockSpec((_R, 9), lambda i: (i, 0)),
                  pl.BlockSpec((9, 16), lambda i: (0, 0)),
                  pl.BlockSpec((1, 16), lambda i: (0, 0)),
                  pl.BlockSpec((16, 96), lambda i: (0, 0)),
                  pl.BlockSpec((1, 96), lambda i: (0, 0)),
                  pl.BlockSpec((16, 32), lambda i: (0, 0)),
                  pl.BlockSpec((1, 32), lambda i: (0, 0))],
        out_specs=pl.BlockSpec((_R, 128), lambda i: (i, 0)),
        out_shape=jax.ShapeDtypeStruct((N, 128), jnp.float32),
    )(e1a, den1, x, ws1, bs1, w2, b2, ws2, bs2)


def _final(e2a, den2, s2, batch, wfc, bfc):
    nstep = N // _R

    def body(a_ref, d_ref, s_ref, b_ref, wfc_ref, bfc_ref, o_ref, gsum_ref,
             cnt_ref):
        i = pl.program_id(0)

        @pl.when(i == 0)
        def _():
            gsum_ref[...] = jnp.zeros_like(gsum_ref)
            cnt_ref[...] = jnp.zeros_like(cnt_ref)

        h = jnp.maximum(a_ref[...] / jnp.maximum(d_ref[...], 1e-16)
                        + s_ref[...], 0.0)
        oh = (b_ref[...] == lax.broadcasted_iota(jnp.int32, (_R, G), 1)
              ).astype(jnp.float32)
        gsum_ref[...] += lax.dot_general(oh, h, (((0,), (0,)), ((), ())),
                                         preferred_element_type=jnp.float32)
        cnt_ref[...] += lax.dot_general(oh, jnp.ones((_R, 8), jnp.float32),
                                        (((0,), (0,)), ((), ())),
                                        preferred_element_type=jnp.float32)

        @pl.when(i == nstep - 1)
        def _():
            gmean = gsum_ref[...] / jnp.maximum(cnt_ref[...][:, 0:1], 1.0)
            o_ref[...] = jnp.maximum(
                jnp.dot(gmean, wfc_ref[...],
                        preferred_element_type=jnp.float32) + bfc_ref[...], 0.0)

    out, _, _ = pl.pallas_call(
        body,
        grid=(nstep,),
        in_specs=[pl.BlockSpec((_R, 32), lambda i: (i, 0)),
                  pl.BlockSpec((_R, 1), lambda i: (i, 0)),
                  pl.BlockSpec((_R, 32), lambda i: (i, 0)),
                  pl.BlockSpec((_R, 1), lambda i: (i, 0)),
                  pl.BlockSpec((32, 2), lambda i: (0, 0)),
                  pl.BlockSpec((1, 2), lambda i: (0, 0))],
        out_specs=[pl.BlockSpec((G, 2), lambda i: (0, 0)),
                   pl.BlockSpec((G, 32), lambda i: (0, 0)),
                   pl.BlockSpec((G, 8), lambda i: (0, 0))],
        out_shape=[jax.ShapeDtypeStruct((G, 2), jnp.float32),
                   jax.ShapeDtypeStruct((G, 32), jnp.float32),
                   jax.ShapeDtypeStruct((G, 8), jnp.float32)],
    )(e2a, den2, s2, batch, wfc, bfc)
    return out


def kernel(x, edge_index, batch, Wq1, bq1, Wk1, bk1, Wv1, bv1, Ws1, bs1,
           Wq2, bq2, Wk2, bk2, Wv2, bv2, Ws2, bs2, Wfc, bfc):
    src = edge_index[0].astype(jnp.int32)
    dst = edge_index[1].astype(jnp.int32)
    pad = jnp.zeros((E_PAD - E,), jnp.int32)
    srcp = jnp.concatenate([src, pad]).reshape(E_PAD // 128, 128)
    dstp = jnp.concatenate([dst, pad]).reshape(E_PAD // 128, 128)

    w1 = jnp.concatenate([Wq1, Wk1, Wv1], axis=1)
    b1 = jnp.concatenate([bq1, bk1, bv1]).reshape(1, 48)
    o1 = _proj1(x, w1, b1)

    e1a, e1d = _edge1(srcp, dstp, o1[:, 0:16], o1[:, 16:32], o1[:, 32:48])
    e1a = e1a.reshape(2, N, 16)
    den1 = e1d.reshape(2, -1)[:, :N].reshape(2, N, 1)

    w2 = jnp.concatenate([Wq2, Wk2, Wv2], axis=1)
    b2 = jnp.concatenate([bq2, bk2, bv2]).reshape(1, 96)
    o2 = _mid(e1a, den1, x, Ws1, bs1.reshape(1, 16), w2, b2, Ws2,
              bs2.reshape(1, 32))

    e2a, e2d = _edge2(srcp, dstp, o2[:, 0:32], o2[:, 32:64], o2[:, 64:96])
    e2a = e2a.reshape(N, 32)
    den2 = e2d.reshape(2, -1)[:, :HALF].reshape(N, 1)

    return _final(e2a, den2, o2[:, 96:128],
                  batch.astype(jnp.int32).reshape(N, 1), Wfc,
                  bfc.reshape(1, 2))
